# baseline (speedup 1.0000x reference)
import functools

import jax
import jax.numpy as jnp
from jax import lax
from jax.experimental import pallas as pl
from jax.experimental.pallas import tpu as pltpu
from jax.experimental.pallas import tpu_sc as plsc

N_NODES = 10000
N_EDGES = 320000
D_IN = 128
D_HID = 128
N_CLS = 32

NW = 32
LANES = 128
CHUNKS = 80
SUP = 10
INNER = CHUNKS // SUP
E_PAD = NW * CHUNKS * LANES
GL = 64
GCHUNKS = 160
GINNER = 16
NBUF = 4
N_PAD = 10240
ROWS_PER_TILE = N_PAD // 16
DEG_W = 16

_mesh = plsc.VectorSubcoreMesh(core_axis_name="c", subcore_axis_name="s")


def _make_sc_agg(feat_dim):
    out_types = [jax.ShapeDtypeStruct((2, N_PAD, feat_dim), jnp.float32)]
    half = LANES // 2
    scratch = [
        pltpu.VMEM((INNER, LANES), jnp.int32),
        pltpu.VMEM((INNER, LANES), jnp.int32),
        pltpu.VMEM((LANES, feat_dim), jnp.float32),
        pltpu.VMEM((LANES, feat_dim), jnp.float32),
        pltpu.VMEM_SHARED((N_PAD, feat_dim), jnp.float32),
        pltpu.SemaphoreType.DMA,
        pltpu.SemaphoreType.DMA,
        pltpu.SemaphoreType.DMA,
        pltpu.SemaphoreType.DMA,
        pltpu.SemaphoreType.DMA,
        pltpu.SemaphoreType.DMA,
    ]

    def body(table_hbm, src_hbm, dst_hbm, zeros_hbm,
             out_hbm, srcv, dstv, rows0, rows1, acc, i0, i1, *gsem):
        rowbuf = (rows0, rows1)
        cid = lax.axis_index("c")
        sid = lax.axis_index("s")
        wid = cid * 16 + sid
        r0 = sid * ROWS_PER_TILE
        pltpu.sync_copy(zeros_hbm, acc.at[pl.ds(r0, ROWS_PER_TILE), :])
        plsc.subcore_barrier()

        def fire(c, p):
            for h in range(2):
                pltpu.async_copy(
                    table_hbm.at[srcv.at[c, pl.ds(h * half, half)]],
                    rowbuf[p].at[pl.ds(h * half, half), :],
                    gsem[p * 2 + h])

        def drain(p):
            for h in range(2):
                pltpu.make_async_copy(
                    table_hbm.at[pl.ds(0, half), :],
                    rowbuf[p].at[pl.ds(h * half, half), :],
                    gsem[p * 2 + h]).wait()

        def super_step(k, carry):
            base = wid * CHUNKS + k * INNER
            ha = pltpu.async_copy(src_hbm.at[pl.ds(base, INNER), :], srcv, i0)
            hb = pltpu.async_copy(dst_hbm.at[pl.ds(base, INNER), :], dstv, i1)
            ha.wait()
            hb.wait()
            fire(0, 0)
            fire(1, 1)
            for j in range(INNER):
                p = j & 1
                drain(p)
                pltpu.sync_copy(rowbuf[p], acc.at[dstv.at[j]], add=True)
                if j + 2 < INNER:
                    fire(j + 2, p)
            return carry

        lax.fori_loop(0, SUP, super_step, 0)
        plsc.subcore_barrier()
        pltpu.sync_copy(acc.at[pl.ds(r0, ROWS_PER_TILE), :],
                        out_hbm.at[cid, pl.ds(r0, ROWS_PER_TILE), :])

    return functools.partial(
        pl.kernel, mesh=_mesh, out_type=out_types, scratch_types=scratch
    )(body)


def _sc_deg_kernel(dst_hbm, zeros_hbm, ones_hbm, deg_hbm,
                   dstv, ones, dacc, sem):
    cid = lax.axis_index("c")
    sid = lax.axis_index("s")
    wid = cid * 16 + sid
    r0 = sid * ROWS_PER_TILE
    pltpu.sync_copy(zeros_hbm, dacc.at[pl.ds(r0, ROWS_PER_TILE), :])
    pltpu.sync_copy(ones_hbm, ones)
    plsc.subcore_barrier()

    def step(j, carry):
        pltpu.sync_copy(ones, dacc.at[dstv.at[j]], add=True)
        return carry

    def super_step(k, carry):
        base = wid * CHUNKS + k * INNER
        pltpu.sync_copy(dst_hbm.at[pl.ds(base, INNER), :], dstv)
        lax.fori_loop(0, INNER, step, 0)
        return carry

    lax.fori_loop(0, SUP, super_step, 0)
    plsc.subcore_barrier()
    pltpu.sync_copy(dacc.at[pl.ds(r0, ROWS_PER_TILE), :],
                    deg_hbm.at[cid, pl.ds(r0, ROWS_PER_TILE), :])


_sc_deg = functools.partial(
    pl.kernel, mesh=_mesh,
    out_type=[jax.ShapeDtypeStruct((2, N_PAD, D_IN), jnp.float32)],
    scratch_types=[
        pltpu.VMEM((INNER, LANES), jnp.int32),
        pltpu.VMEM((LANES, D_IN), jnp.float32),
        pltpu.VMEM_SHARED((N_PAD, D_IN), jnp.float32),
        pltpu.SemaphoreType.DMA,
    ],
)(_sc_deg_kernel)


_sc_agg1 = _make_sc_agg(D_IN)
_sc_agg2 = _make_sc_agg(D_HID)

_BLK = 1000
_GRID = N_NODES // _BLK


def _tc1_body(x_ref, s1_ref, deg_ref, ws1_ref, wn1_ref, b1_ref,
              ws2_ref, h_ref, q_ref):
    s = s1_ref[0] + s1_ref[1]
    dg = deg_ref[0] + deg_ref[1]
    hn = s / jnp.maximum(dg, 1.0)
    h1 = jnp.maximum(
        x_ref[...] @ ws1_ref[...] + hn @ wn1_ref[...] + b1_ref[...], 0.0)
    h_ref[...] = h1
    q_ref[...] = h1 @ ws2_ref[...]


def _tc2_body(q_ref, s2_ref, deg_ref, wn2_ref, b2_ref, o_ref):
    s = s2_ref[0] + s2_ref[1]
    dg = deg_ref[0] + deg_ref[1]
    hn = s / jnp.maximum(dg, 1.0)
    o_ref[...] = jax.nn.sigmoid(
        q_ref[...] + hn @ wn2_ref[...] + b2_ref[...])


_tc1 = pl.pallas_call(
    _tc1_body,
    grid=(_GRID,),
    in_specs=[
        pl.BlockSpec((_BLK, D_IN), lambda i: (i, 0)),
        pl.BlockSpec((2, _BLK, D_IN), lambda i: (0, i, 0)),
        pl.BlockSpec((2, _BLK, D_IN), lambda i: (0, i, 0)),
        pl.BlockSpec((D_IN, D_HID), lambda i: (0, 0)),
        pl.BlockSpec((D_IN, D_HID), lambda i: (0, 0)),
        pl.BlockSpec((1, D_HID), lambda i: (0, 0)),
        pl.BlockSpec((D_HID, N_CLS), lambda i: (0, 0)),
    ],
    out_specs=[
        pl.BlockSpec((_BLK, D_HID), lambda i: (i, 0)),
        pl.BlockSpec((_BLK, N_CLS), lambda i: (i, 0)),
    ],
    out_shape=[
        jax.ShapeDtypeStruct((N_NODES, D_HID), jnp.float32),
        jax.ShapeDtypeStruct((N_NODES, N_CLS), jnp.float32),
    ],
)

_tc2 = pl.pallas_call(
    _tc2_body,
    grid=(_GRID,),
    in_specs=[
        pl.BlockSpec((_BLK, N_CLS), lambda i: (i, 0)),
        pl.BlockSpec((2, _BLK, D_HID), lambda i: (0, i, 0)),
        pl.BlockSpec((2, _BLK, D_IN), lambda i: (0, i, 0)),
        pl.BlockSpec((D_HID, N_CLS), lambda i: (0, 0)),
        pl.BlockSpec((1, N_CLS), lambda i: (0, 0)),
    ],
    out_specs=pl.BlockSpec((_BLK, N_CLS), lambda i: (i, 0)),
    out_shape=jax.ShapeDtypeStruct((N_NODES, N_CLS), jnp.float32),
)


def kernel(x, edge_index, W_self1, W_neigh1, b1, W_self2, W_neigh2, b2):
    src = edge_index[0].astype(jnp.int32)
    dst = edge_index[1].astype(jnp.int32)
    pad = E_PAD - N_EDGES
    srcf = jnp.concatenate([src, jnp.zeros((pad,), jnp.int32)])
    dstf = jnp.concatenate([dst, jnp.full((pad,), N_NODES, jnp.int32)])
    srcp = srcf.reshape(NW * CHUNKS, LANES)
    dstp = dstf.reshape(NW * CHUNKS, LANES)
    dstp128 = dstp
    zeros_w = jnp.zeros((ROWS_PER_TILE, D_IN), jnp.float32)
    ones_w = jnp.ones((LANES, D_IN), jnp.float32)

    (deg,) = _sc_deg(dstp128, zeros_w, ones_w)
    (s1,) = _sc_agg1(x, srcp, dstp, zeros_w)
    h1, q2 = _tc1(x, s1[:, :N_NODES, :], deg[:, :N_NODES, :],
                  W_self1, W_neigh1, b1.reshape(1, D_HID), W_self2)
    (s2,) = _sc_agg2(h1, srcp, dstp, zeros_w)
    return _tc2(q2, s2[:, :N_NODES, :], deg[:, :N_NODES, :],
                W_neigh2, b2.reshape(1, N_CLS))


# speedup vs baseline: 1.0611x; 1.0611x over previous
import functools

import jax
import jax.numpy as jnp
from jax import lax
from jax.experimental import pallas as pl
from jax.experimental.pallas import tpu as pltpu
from jax.experimental.pallas import tpu_sc as plsc

N_NODES = 10000
N_EDGES = 320000
D_IN = 128
D_HID = 128
N_CLS = 32

NW = 32
LANES = 128
CHUNKS = 80
SUP = 10
INNER = CHUNKS // SUP
E_PAD = NW * CHUNKS * LANES
GL = 64
GCHUNKS = 160
GINNER = 16
NBUF = 4
N_PAD = 10240
ROWS_PER_TILE = N_PAD // 16
DEG_W = 16

_mesh = plsc.VectorSubcoreMesh(core_axis_name="c", subcore_axis_name="s")


def _make_sc_agg(feat_dim):
    out_types = [jax.ShapeDtypeStruct((2, N_PAD, feat_dim), jnp.float32)]
    half = LANES // 2
    scratch = [
        pltpu.VMEM((INNER, LANES), jnp.int32),
        pltpu.VMEM((INNER, LANES), jnp.int32),
        pltpu.VMEM((LANES, feat_dim), jnp.float32),
        pltpu.VMEM((LANES, feat_dim), jnp.float32),
        pltpu.VMEM_SHARED((N_PAD, feat_dim), jnp.float32),
        pltpu.SemaphoreType.DMA,
        pltpu.SemaphoreType.DMA,
        pltpu.SemaphoreType.DMA,
        pltpu.SemaphoreType.DMA,
        pltpu.SemaphoreType.DMA,
        pltpu.SemaphoreType.DMA,
    ]

    def body(table_hbm, src_hbm, dst_hbm, zeros_hbm,
             out_hbm, srcv, dstv, rows0, rows1, acc, i0, i1, *gsem):
        rowbuf = (rows0, rows1)
        cid = lax.axis_index("c")
        sid = lax.axis_index("s")
        wid = cid * 16 + sid
        r0 = sid * ROWS_PER_TILE
        pltpu.sync_copy(zeros_hbm, acc.at[pl.ds(r0, ROWS_PER_TILE), :])
        plsc.subcore_barrier()

        def fire(c, p):
            for h in range(2):
                pltpu.async_copy(
                    table_hbm.at[srcv.at[c, pl.ds(h * half, half)]],
                    rowbuf[p].at[pl.ds(h * half, half), :],
                    gsem[p * 2 + h])

        def drain(p):
            for h in range(2):
                pltpu.make_async_copy(
                    table_hbm.at[pl.ds(0, half), :],
                    rowbuf[p].at[pl.ds(h * half, half), :],
                    gsem[p * 2 + h]).wait()

        def super_step(k, carry):
            base = wid * CHUNKS + k * INNER
            ha = pltpu.async_copy(src_hbm.at[pl.ds(base, INNER), :], srcv, i0)
            hb = pltpu.async_copy(dst_hbm.at[pl.ds(base, INNER), :], dstv, i1)
            ha.wait()
            hb.wait()
            fire(0, 0)
            fire(1, 1)
            for j in range(INNER):
                p = j & 1
                drain(p)
                pltpu.sync_copy(rowbuf[p], acc.at[dstv.at[j]], add=True)
                if j + 2 < INNER:
                    fire(j + 2, p)
            return carry

        lax.fori_loop(0, SUP, super_step, 0)
        plsc.subcore_barrier()
        pltpu.sync_copy(acc.at[pl.ds(r0, ROWS_PER_TILE), :],
                        out_hbm.at[cid, pl.ds(r0, ROWS_PER_TILE), :])

    return functools.partial(
        pl.kernel, mesh=_mesh, out_type=out_types, scratch_types=scratch
    )(body)


def _sc_deg_kernel(dst_hbm, zeros_hbm, ones_hbm, deg_hbm,
                   dstv, ones, dacc, sem):
    cid = lax.axis_index("c")
    sid = lax.axis_index("s")
    wid = cid * 16 + sid
    r0 = sid * ROWS_PER_TILE
    pltpu.sync_copy(zeros_hbm, dacc.at[pl.ds(r0, ROWS_PER_TILE), :])
    pltpu.sync_copy(ones_hbm, ones)
    plsc.subcore_barrier()

    def step(j, carry):
        pltpu.sync_copy(ones, dacc.at[dstv.at[j]], add=True)
        return carry

    def super_step(k, carry):
        base = wid * CHUNKS + k * INNER
        pltpu.sync_copy(dst_hbm.at[pl.ds(base, INNER), :], dstv)
        lax.fori_loop(0, INNER, step, 0)
        return carry

    lax.fori_loop(0, SUP, super_step, 0)
    plsc.subcore_barrier()
    pltpu.sync_copy(dacc.at[pl.ds(r0, ROWS_PER_TILE), :],
                    deg_hbm.at[cid, pl.ds(r0, ROWS_PER_TILE), :])


_sc_deg = functools.partial(
    pl.kernel, mesh=_mesh,
    out_type=[jax.ShapeDtypeStruct((2, N_PAD, D_IN), jnp.float32)],
    scratch_types=[
        pltpu.VMEM((INNER, LANES), jnp.int32),
        pltpu.VMEM((LANES, D_IN), jnp.float32),
        pltpu.VMEM_SHARED((N_PAD, D_IN), jnp.float32),
        pltpu.SemaphoreType.DMA,
    ],
)(_sc_deg_kernel)


_sc_agg1 = _make_sc_agg(D_IN)
_sc_agg2 = _make_sc_agg(D_HID)

_BLK = 1000
_GRID = N_NODES // _BLK


def _tc1_body(x_ref, s1_ref, deg_ref, ws1_ref, wn1_ref, b1_ref,
              ws2_ref, h_ref, q_ref):
    s = s1_ref[0] + s1_ref[1]
    dg = deg_ref[0] + deg_ref[1]
    hn = s / jnp.maximum(dg, 1.0)
    h1 = jnp.maximum(
        x_ref[...] @ ws1_ref[...] + hn @ wn1_ref[...] + b1_ref[...], 0.0)
    h_ref[...] = h1
    q_ref[...] = h1 @ ws2_ref[...]


def _tc2_body(q_ref, s2_ref, deg_ref, wn2_ref, b2_ref, o_ref):
    s = s2_ref[0] + s2_ref[1]
    dg = deg_ref[0] + deg_ref[1]
    hn = s / jnp.maximum(dg, 1.0)
    o_ref[...] = jax.nn.sigmoid(
        q_ref[...] + hn @ wn2_ref[...] + b2_ref[...])


_tc1 = pl.pallas_call(
    _tc1_body,
    grid=(_GRID,),
    in_specs=[
        pl.BlockSpec((_BLK, D_IN), lambda i: (i, 0)),
        pl.BlockSpec((2, _BLK, D_IN), lambda i: (0, i, 0)),
        pl.BlockSpec((2, _BLK, D_IN), lambda i: (0, i, 0)),
        pl.BlockSpec((D_IN, D_HID), lambda i: (0, 0)),
        pl.BlockSpec((D_IN, D_HID), lambda i: (0, 0)),
        pl.BlockSpec((1, D_HID), lambda i: (0, 0)),
        pl.BlockSpec((D_HID, N_CLS), lambda i: (0, 0)),
    ],
    out_specs=[
        pl.BlockSpec((_BLK, D_HID), lambda i: (i, 0)),
        pl.BlockSpec((_BLK, N_CLS), lambda i: (i, 0)),
    ],
    out_shape=[
        jax.ShapeDtypeStruct((N_NODES, D_HID), jnp.float32),
        jax.ShapeDtypeStruct((N_NODES, N_CLS), jnp.float32),
    ],
)

_tc2 = pl.pallas_call(
    _tc2_body,
    grid=(_GRID,),
    in_specs=[
        pl.BlockSpec((_BLK, N_CLS), lambda i: (i, 0)),
        pl.BlockSpec((2, _BLK, D_HID), lambda i: (0, i, 0)),
        pl.BlockSpec((2, _BLK, D_IN), lambda i: (0, i, 0)),
        pl.BlockSpec((D_HID, N_CLS), lambda i: (0, 0)),
        pl.BlockSpec((1, N_CLS), lambda i: (0, 0)),
    ],
    out_specs=pl.BlockSpec((_BLK, N_CLS), lambda i: (i, 0)),
    out_shape=jax.ShapeDtypeStruct((N_NODES, N_CLS), jnp.float32),
)


def kernel(x, edge_index, W_self1, W_neigh1, b1, W_self2, W_neigh2, b2):
    src = edge_index[0].astype(jnp.int32)
    dst = edge_index[1].astype(jnp.int32)
    pad = E_PAD - N_EDGES
    srcf = jnp.concatenate([src, jnp.zeros((pad,), jnp.int32)])
    dstf = jnp.concatenate([dst, jnp.full((pad,), N_NODES, jnp.int32)])
    srcp = srcf.reshape(NW * CHUNKS, LANES)
    dstp = dstf.reshape(NW * CHUNKS, LANES)
    dstp128 = dstp
    zeros_w = jnp.zeros((ROWS_PER_TILE, D_IN), jnp.float32)
    ones_w = jnp.ones((LANES, D_IN), jnp.float32)

    (deg,) = _sc_deg(dstp128, zeros_w, ones_w)
    (s1,) = _sc_agg1(x, srcp, dstp, zeros_w)
    h1, q2 = _tc1(x, s1, deg, W_self1, W_neigh1, b1.reshape(1, D_HID), W_self2)
    (s2,) = _sc_agg2(h1, srcp, dstp, zeros_w)
    return _tc2(q2, s2, deg, W_neigh2, b2.reshape(1, N_CLS))
